# Initial kernel scaffold; baseline (speedup 1.0000x reference)
import jax
import jax.numpy as jnp
from jax.experimental import pallas as pl


def kernel(x, edge_index, batch, W0, b0, Ws, bs, gammas, betas, Wf, bf):
    raise NotImplementedError("write your pallas kernel here")


# speedup vs baseline: 6.3842x; 6.3842x over previous
import functools

import jax
import jax.numpy as jnp
from jax import lax
from jax.experimental import pallas as pl
from jax.experimental.pallas import tpu as pltpu
from jax.experimental.pallas import tpu_sc as plsc

_N = 10000
_E = 320000
_D = 128
_L = 8
_G = 16

_NP = 10240
_NSC = 2
_NT = 16
_CH = 80
_EPT = _E // (_NSC * _NT)
_NCH = _EPT // _CH
_RPT = _NP // _NT
_BN = 1024

_sc_mesh = plsc.VectorSubcoreMesh(core_axis_name="c", subcore_axis_name="s")


@functools.partial(
    pl.kernel,
    out_type=jax.ShapeDtypeStruct((_NSC, _NP, _D), jnp.float32),
    mesh=_sc_mesh,
    scratch_types=[
        pltpu.VMEM((_CH,), jnp.int32),
        pltpu.VMEM((_CH,), jnp.int32),
        pltpu.VMEM((_CH, _D), jnp.float32),
        pltpu.VMEM_SHARED((_NP, _D), jnp.float32),
        pltpu.SemaphoreType.DMA,
    ],
)
def _sc_agg(src_hbm, dst_hbm, hw_hbm, zeros_hbm, out_hbm, srcv, dstv, rows,
            acc, sem):
    c = lax.axis_index("c")
    s = lax.axis_index("s")
    wid = c * _NT + s
    pltpu.sync_copy(zeros_hbm, rows)
    row0 = s * _RPT
    for j in range(_RPT // _CH):
        pltpu.sync_copy(rows, acc.at[pl.ds(row0 + j * _CH, _CH)])
    plsc.subcore_barrier()
    ebase = wid * _EPT

    def body(k, carry):
        eo = ebase + k * _CH
        pltpu.sync_copy(src_hbm.at[pl.ds(eo, _CH)], srcv)
        pltpu.sync_copy(dst_hbm.at[pl.ds(eo, _CH)], dstv)
        pltpu.async_copy(hw_hbm.at[srcv], rows, sem).wait()
        pltpu.sync_copy(rows, acc.at[dstv], add=True)
        return carry

    lax.fori_loop(0, _NCH, body, 0)
    plsc.subcore_barrier()
    for j in range(_RPT // _CH):
        r = row0 + j * _CH
        pltpu.sync_copy(acc.at[pl.ds(r, _CH)], rows)
        pltpu.sync_copy(rows, out_hbm.at[c, pl.ds(r, _CH)])


@functools.partial(
    pl.kernel,
    out_type=jax.ShapeDtypeStruct((_NSC, _NP, _D), jnp.float32),
    mesh=_sc_mesh,
    scratch_types=[
        pltpu.VMEM((_CH,), jnp.int32),
        pltpu.VMEM((_CH, _D), jnp.float32),
        pltpu.VMEM((_CH, _D), jnp.float32),
        pltpu.VMEM_SHARED((_NP, _D), jnp.float32),
    ],
)
def _sc_deg(dst_hbm, ones_hbm, zeros_hbm, out_hbm, dstv, onesv, stage, acc):
    c = lax.axis_index("c")
    s = lax.axis_index("s")
    wid = c * _NT + s
    pltpu.sync_copy(ones_hbm, onesv)
    pltpu.sync_copy(zeros_hbm, stage)
    row0 = s * _RPT
    for j in range(_RPT // _CH):
        pltpu.sync_copy(stage, acc.at[pl.ds(row0 + j * _CH, _CH)])
    plsc.subcore_barrier()
    ebase = wid * _EPT

    def body(k, carry):
        pltpu.sync_copy(dst_hbm.at[pl.ds(ebase + k * _CH, _CH)], dstv)
        pltpu.sync_copy(onesv, acc.at[dstv], add=True)
        return carry

    lax.fori_loop(0, _NCH, body, 0)
    plsc.subcore_barrier()
    for j in range(_RPT // _CH):
        r = row0 + j * _CH
        pltpu.sync_copy(acc.at[pl.ds(r, _CH)], stage)
        pltpu.sync_copy(stage, out_hbm.at[c, pl.ds(r, _CH)])


def _tc_pre_body(x_ref, deg_ref, W0_ref, b0_ref, Ws0_ref, dis_ref, hw_ref):
    counts = deg_ref[0][:, 0:1] + deg_ref[1][:, 0:1]
    dis = lax.rsqrt(counts + 1.0)
    h0 = jnp.dot(x_ref[...], W0_ref[...], preferred_element_type=jnp.float32)
    h0 = h0 + b0_ref[...]
    hw = jnp.dot(h0, Ws0_ref[...], preferred_element_type=jnp.float32) * dis
    dis_ref[...] = dis
    hw_ref[...] = hw


_tc_pre = pl.pallas_call(
    _tc_pre_body,
    grid=(_NP // _BN,),
    in_specs=[
        pl.BlockSpec((_BN, _D), lambda i: (i, 0)),
        pl.BlockSpec((_NSC, _BN, _D), lambda i: (0, i, 0)),
        pl.BlockSpec((_D, _D), lambda i: (0, 0)),
        pl.BlockSpec((1, _D), lambda i: (0, 0)),
        pl.BlockSpec((_D, _D), lambda i: (0, 0)),
    ],
    out_specs=[
        pl.BlockSpec((_BN, 1), lambda i: (i, 0)),
        pl.BlockSpec((_BN, _D), lambda i: (i, 0)),
    ],
    out_shape=[
        jax.ShapeDtypeStruct((_NP, 1), jnp.float32),
        jax.ShapeDtypeStruct((_NP, _D), jnp.float32),
    ],
)


def _combine_ln_relu(acc_ref, hw_ref, dis_ref, b_ref, g_ref, be_ref):
    dis = dis_ref[...]
    t = (acc_ref[0] + acc_ref[1] + hw_ref[...]) * dis + b_ref[...]
    mu = jnp.mean(t, axis=-1, keepdims=True)
    d = t - mu
    var = jnp.mean(d * d, axis=-1, keepdims=True)
    tn = d * lax.rsqrt(var + 1e-5) * g_ref[...] + be_ref[...]
    return jnp.maximum(tn, 0.0)


def _tc_mid_body(acc_ref, hw_ref, dis_ref, b_ref, g_ref, be_ref, Wn_ref,
                 out_ref):
    h = _combine_ln_relu(acc_ref, hw_ref, dis_ref, b_ref, g_ref, be_ref)
    out_ref[...] = (jnp.dot(h, Wn_ref[...], preferred_element_type=jnp.float32)
                    * dis_ref[...])


_tc_mid = pl.pallas_call(
    _tc_mid_body,
    grid=(_NP // _BN,),
    in_specs=[
        pl.BlockSpec((_NSC, _BN, _D), lambda i: (0, i, 0)),
        pl.BlockSpec((_BN, _D), lambda i: (i, 0)),
        pl.BlockSpec((_BN, 1), lambda i: (i, 0)),
        pl.BlockSpec((1, _D), lambda i: (0, 0)),
        pl.BlockSpec((1, _D), lambda i: (0, 0)),
        pl.BlockSpec((1, _D), lambda i: (0, 0)),
        pl.BlockSpec((_D, _D), lambda i: (0, 0)),
    ],
    out_specs=pl.BlockSpec((_BN, _D), lambda i: (i, 0)),
    out_shape=jax.ShapeDtypeStruct((_NP, _D), jnp.float32),
)


def _tc_last_body(acc_ref, hw_ref, dis_ref, b_ref, g_ref, be_ref, batch_ref,
                  Wf_ref, bf_ref, out_ref, pool_ref, cnt_ref):
    i = pl.program_id(0)

    @pl.when(i == 0)
    def _():
        pool_ref[...] = jnp.zeros_like(pool_ref)
        cnt_ref[...] = jnp.zeros_like(cnt_ref)

    h = _combine_ln_relu(acc_ref, hw_ref, dis_ref, b_ref, g_ref, be_ref)
    onehot = (batch_ref[...] ==
              lax.broadcasted_iota(jnp.int32, (1, _G), 1)).astype(jnp.float32)
    pool_ref[...] += lax.dot_general(
        onehot, h, (((0,), (0,)), ((), ())),
        preferred_element_type=jnp.float32)
    cnt_ref[...] += lax.dot_general(
        onehot, jnp.ones((_BN, _D), jnp.float32), (((0,), (0,)), ((), ())),
        preferred_element_type=jnp.float32)

    @pl.when(i == pl.num_programs(0) - 1)
    def _():
        pooled = pool_ref[...] / jnp.maximum(cnt_ref[...], 1.0)
        out_ref[...] = (jnp.dot(pooled, Wf_ref[...],
                                preferred_element_type=jnp.float32)
                        + bf_ref[...])


_tc_last = pl.pallas_call(
    _tc_last_body,
    grid=(_NP // _BN,),
    in_specs=[
        pl.BlockSpec((_NSC, _BN, _D), lambda i: (0, i, 0)),
        pl.BlockSpec((_BN, _D), lambda i: (i, 0)),
        pl.BlockSpec((_BN, 1), lambda i: (i, 0)),
        pl.BlockSpec((1, _D), lambda i: (0, 0)),
        pl.BlockSpec((1, _D), lambda i: (0, 0)),
        pl.BlockSpec((1, _D), lambda i: (0, 0)),
        pl.BlockSpec((_BN, 1), lambda i: (i, 0)),
        pl.BlockSpec((_D, _D), lambda i: (0, 0)),
        pl.BlockSpec((1, _D), lambda i: (0, 0)),
    ],
    out_specs=pl.BlockSpec((_G, _D), lambda i: (0, 0)),
    out_shape=jax.ShapeDtypeStruct((_G, _D), jnp.float32),
    scratch_shapes=[
        pltpu.VMEM((_G, _D), jnp.float32),
        pltpu.VMEM((_G, _D), jnp.float32),
    ],
)


def kernel(x, edge_index, batch, W0, b0, Ws, bs, gammas, betas, Wf, bf):
    src = edge_index[0]
    dst = edge_index[1]
    xp = jnp.zeros((_NP, _D), jnp.float32).at[:_N].set(x)
    batch_p = jnp.full((_NP, 1), _G, jnp.int32).at[:_N, 0].set(batch)
    zeros_row = jnp.zeros((_CH, _D), jnp.float32)
    ones_row = jnp.ones((_CH, _D), jnp.float32)

    deg = _sc_deg(dst, ones_row, zeros_row)
    dis, hw = _tc_pre(xp, deg, W0, b0[None, :], Ws[0])
    out = None
    for i in range(_L):
        accs = _sc_agg(src, dst, hw, zeros_row)
        if i < _L - 1:
            hw = _tc_mid(accs, hw, dis, bs[i][None, :], gammas[i][None, :],
                         betas[i][None, :], Ws[i + 1])
        else:
            out = _tc_last(accs, hw, dis, bs[i][None, :], gammas[i][None, :],
                           betas[i][None, :], batch_p, Wf, bf[None, :])
    return out
